# baseline (speedup 1.0000x reference)
import functools

import jax
import jax.numpy as jnp
from jax import lax
from jax.experimental import pallas as pl
from jax.experimental.pallas import tpu as pltpu
from jax.experimental.pallas import tpu_sc as plsc

_N, _K, _D = 16384, 1024, 16
_L = 16
_NC, _NS = 2, 16
_NW = _NC * _NS
_RPW = _N // _NW


def _sqrt16(x):
    xc = jnp.maximum(x, 1e-12)
    s = 0.5 * (1.0 + xc)
    for _ in range(8):
        s = 0.5 * (s + xc / s)
    return s


_mesh = plsc.VectorSubcoreMesh(core_axis_name="c", subcore_axis_name="s")


@functools.partial(
    pl.kernel,
    mesh=_mesh,
    out_type=jax.ShapeDtypeStruct((_NW, _L), jnp.float32),
    scratch_types=[
        pltpu.VMEM((_D, _K), jnp.float32),
        pltpu.VMEM((_K,), jnp.float32),
        pltpu.VMEM((_RPW, _D), jnp.float32),
        pltpu.VMEM((1, _L), jnp.float32),
    ],
)
def _sc_kernel(ct_hbm, f_hbm, out_hbm, ct_v, csq_v, f_v, acc_v):
    wid = lax.axis_index("s") * _NC + lax.axis_index("c")
    pltpu.sync_copy(ct_hbm, ct_v)
    pltpu.sync_copy(f_hbm.at[pl.ds(wid * _RPW, _RPW)], f_v)

    def csq_body(g, carry):
        a = jnp.zeros((_L,), jnp.float32)
        for d in range(_D):
            cv = ct_v[d, pl.ds(g * _L, _L)]
            a = a + cv * cv
        csq_v[pl.ds(g * _L, _L)] = a
        return carry

    lax.fori_loop(0, _K // _L, csq_body, 0)

    lane = lax.iota(jnp.int32, _L)
    big = jnp.full((_L,), 1e30, jnp.float32)

    def block_body(b, acc):
        def pair_body(q, stage):
            i0 = (b * 8 + q) * 2
            ra = f_v[i0, pl.ds(0, _D)]
            rb = f_v[i0 + 1, pl.ds(0, _D)]
            ea = [ra[d] for d in range(_D)]
            eb = [rb[d] for d in range(_D)]
            fsq_a = ea[0] * ea[0]
            fsq_b = eb[0] * eb[0]
            for d in range(1, _D):
                fsq_a = fsq_a + ea[d] * ea[d]
                fsq_b = fsq_b + eb[d] * eb[d]
            sa = [jnp.full((_L,), ea[d] * -2.0, jnp.float32)
                  for d in range(_D)]
            sb = [jnp.full((_L,), eb[d] * -2.0, jnp.float32)
                  for d in range(_D)]

            def g_body(g, mm):
                ma, mb = mm
                base = csq_v[pl.ds(g * _L, _L)]
                acc_a = base
                acc_b = base
                for d in range(_D):
                    cv = ct_v[d, pl.ds(g * _L, _L)]
                    acc_a = acc_a + sa[d] * cv
                    acc_b = acc_b + sb[d] * cv
                return (jnp.minimum(ma, acc_a), jnp.minimum(mb, acc_b))

            ma, mb = lax.fori_loop(0, _K // _L, g_body, (big, big))
            m_a = ma[0]
            m_b = mb[0]
            for j in range(1, _L):
                m_a = jnp.minimum(m_a, ma[j])
                m_b = jnp.minimum(m_b, mb[j])
            m_a = m_a + fsq_a
            m_b = m_b + fsq_b
            stage = jnp.where(lane == 2 * q, jnp.full((_L,), m_a), stage)
            stage = jnp.where(lane == 2 * q + 1, jnp.full((_L,), m_b), stage)
            return stage

        stage = lax.fori_loop(0, 8, pair_body, jnp.zeros((_L,), jnp.float32))
        dist = jnp.minimum(_sqrt16(jnp.maximum(stage, 0.0)), 1000000.0)
        return acc + dist

    acc = lax.fori_loop(0, _RPW // _L, block_body,
                        jnp.zeros((_L,), jnp.float32))
    acc_v[0, :] = acc
    pltpu.sync_copy(acc_v, out_hbm.at[pl.ds(wid, 1)])


def kernel(features, centers):
    ct = centers.T
    partials = _sc_kernel(ct, features)
    return jnp.sum(partials) * (1.0 / features.shape[0])


# speedup vs baseline: 7.0612x; 7.0612x over previous
import functools

import jax
import jax.numpy as jnp
from jax import lax
from jax.experimental import pallas as pl
from jax.experimental.pallas import tpu as pltpu
from jax.experimental.pallas import tpu_sc as plsc

_N, _K, _D = 16384, 1024, 16
_L = 16
_NC, _NS = 2, 16
_NW = _NC * _NS
_SC_ROWS = 512
_RPW = _SC_ROWS // _NW


def _sqrt16(x):
    xc = jnp.maximum(x, 1e-12)
    s = 0.5 * (1.0 + xc)
    for _ in range(8):
        s = 0.5 * (s + xc / s)
    return s


_mesh = plsc.VectorSubcoreMesh(core_axis_name="c", subcore_axis_name="s")


@functools.partial(
    pl.kernel,
    mesh=_mesh,
    out_type=jax.ShapeDtypeStruct((_NW, _L), jnp.float32),
    scratch_types=[
        pltpu.VMEM((_D, _K), jnp.float32),
        pltpu.VMEM((_K,), jnp.float32),
        pltpu.VMEM((_RPW, _D), jnp.float32),
        pltpu.VMEM((1, _L), jnp.float32),
    ],
)
def _sc_kernel(ct_hbm, f_hbm, out_hbm, ct_v, csq_v, f_v, acc_v):
    wid = lax.axis_index("s") * _NC + lax.axis_index("c")
    pltpu.sync_copy(ct_hbm, ct_v)
    pltpu.sync_copy(f_hbm.at[pl.ds(wid * _RPW, _RPW)], f_v)

    def csq_body(g, carry):
        a = jnp.zeros((_L,), jnp.float32)
        for d in range(_D):
            cv = ct_v[d, pl.ds(g * _L, _L)]
            a = a + cv * cv
        csq_v[pl.ds(g * _L, _L)] = a
        return carry

    lax.fori_loop(0, _K // _L, csq_body, 0)

    lane = lax.iota(jnp.int32, _L)
    big = jnp.full((_L,), 1e30, jnp.float32)

    def block_body(b, acc):
        def pair_body(q, stage):
            i0 = (b * 8 + q) * 2
            ra = f_v[i0, pl.ds(0, _D)]
            rb = f_v[i0 + 1, pl.ds(0, _D)]
            ea = [ra[d] for d in range(_D)]
            eb = [rb[d] for d in range(_D)]
            fsq_a = ea[0] * ea[0]
            fsq_b = eb[0] * eb[0]
            for d in range(1, _D):
                fsq_a = fsq_a + ea[d] * ea[d]
                fsq_b = fsq_b + eb[d] * eb[d]
            sa = [jnp.full((_L,), ea[d] * -2.0, jnp.float32)
                  for d in range(_D)]
            sb = [jnp.full((_L,), eb[d] * -2.0, jnp.float32)
                  for d in range(_D)]

            def g_body(g, mm):
                ma, mb = mm
                base = csq_v[pl.ds(g * _L, _L)]
                acc_a = base
                acc_b = base
                for d in range(_D):
                    cv = ct_v[d, pl.ds(g * _L, _L)]
                    acc_a = acc_a + sa[d] * cv
                    acc_b = acc_b + sb[d] * cv
                return (jnp.minimum(ma, acc_a), jnp.minimum(mb, acc_b))

            ma, mb = lax.fori_loop(0, _K // _L, g_body, (big, big))
            m_a = ma[0]
            m_b = mb[0]
            for j in range(1, _L):
                m_a = jnp.minimum(m_a, ma[j])
                m_b = jnp.minimum(m_b, mb[j])
            m_a = m_a + fsq_a
            m_b = m_b + fsq_b
            stage = jnp.where(lane == 2 * q, jnp.full((_L,), m_a), stage)
            stage = jnp.where(lane == 2 * q + 1, jnp.full((_L,), m_b), stage)
            return stage

        stage = lax.fori_loop(0, 8, pair_body, jnp.zeros((_L,), jnp.float32))
        dist = jnp.minimum(_sqrt16(jnp.maximum(stage, 0.0)), 1000000.0)
        return acc + dist

    acc = lax.fori_loop(0, _RPW // _L, block_body,
                        jnp.zeros((_L,), jnp.float32))
    acc_v[0, :] = acc
    pltpu.sync_copy(acc_v, out_hbm.at[pl.ds(wid, 1)])


def _tc_body(ft_ref, c_ref, out_ref, cb_ref):
    i = pl.program_id(0)

    @pl.when(i == 0)
    def _():
        c = c_ref[...]
        csq = jnp.sum(c * c, axis=1, keepdims=True)
        csq_hi = csq.astype(jnp.bfloat16)
        csq_lo = (csq - csq_hi.astype(jnp.float32)).astype(jnp.bfloat16)
        cb_ref[...] = jnp.concatenate(
            [(c * -2.0).astype(jnp.bfloat16), csq_hi, csq_lo], axis=1)
        out_ref[0, 0] = 0.0

    ft = ft_ref[...]
    fsq = jnp.sum(ft * ft, axis=0, keepdims=True)
    fb = ft.astype(jnp.bfloat16)
    ones2 = jnp.ones((2, ft.shape[1]), jnp.bfloat16)
    faug = jnp.concatenate([fb, ones2], axis=0)
    kb = 256
    parts = []
    for j in range(cb_ref.shape[0] // kb):
        sqc = jax.lax.dot_general(
            cb_ref[pl.ds(j * kb, kb), :], faug, (((1,), (0,)), ((), ())),
            preferred_element_type=jnp.float32)
        parts.append(jnp.min(sqc, axis=0, keepdims=True))
    minsq = jnp.min(jnp.concatenate(parts, axis=0), axis=0, keepdims=True)
    dist = jnp.minimum(jnp.sqrt(jnp.maximum(minsq + fsq, 0.0)), 1000000.0)
    out_ref[0, 0] += jnp.sum(dist)


def _tc_call(ft, centers):
    d, bn = ft.shape
    k = centers.shape[0]
    return pl.pallas_call(
        _tc_body,
        grid=(1,),
        in_specs=[
            pl.BlockSpec((d, bn), lambda i: (0, i)),
            pl.BlockSpec((k, d), lambda i: (0, 0)),
        ],
        out_specs=pl.BlockSpec((1, 1), lambda i: (0, 0),
                               memory_space=pltpu.SMEM),
        out_shape=jax.ShapeDtypeStruct((1, 1), jnp.float32),
        scratch_shapes=[pltpu.VMEM((k, d + 2), jnp.bfloat16)],
    )(ft, centers)


def kernel(features, centers):
    n = features.shape[0]
    ct = centers.T
    f_sc = features[:_SC_ROWS]
    ft_tc = features[_SC_ROWS:].T
    sc_partials = _sc_kernel(ct, f_sc)
    tc_sum = _tc_call(ft_tc, centers)
    return (tc_sum[0, 0] + jnp.sum(sc_partials)) * (1.0 / n)
